# baseline (speedup 1.0000x reference)
import functools

import jax
import jax.numpy as jnp
from jax import lax
from jax.experimental import pallas as pl
from jax.experimental.pallas import tpu as pltpu
from jax.experimental.pallas import tpu_sc as plsc

NX = 432
NY = 496
C = 64
HALF = C // 2
P = 2048
B = 2
GRID_N = NY * NX
NROWS = B * C
N_TILES = 32
ROWS_PER_TILE = NROWS // N_TILES
CHUNKS_PER_ROW = 4
CHUNK = GRID_N // CHUNKS_PER_ROW
STEPS = ROWS_PER_TILE * CHUNKS_PER_ROW


def _tc_body(feats_ref, cpack_ref, vecs_ref, wq_ref, wk_ref, wv_ref,
             w1_ref, w2_ref, upd_ref, idx_ref):
    f = feats_ref[0]
    cp = cpack_ref[0]
    iz = cp[:, 1:2]
    iy = cp[:, 2:3]
    ix = cp[:, 3:4]
    ix = ix + iz
    idx_ref[...] = jnp.concatenate([iy, ix, iy, ix, iy, ix, iy, ix], axis=1)

    ar = lax.broadcasted_iota(jnp.int32, (1, HALF), 1).astype(jnp.float32)
    theta = jnp.exp(ar * (-jnp.log(10000.0) / HALF))
    hval = iy.astype(jnp.float32) * (2.0 / (NY - 1)) - 1.0
    wval = ix.astype(jnp.float32) * (2.0 / (NX - 1)) - 1.0
    ca = jnp.cos((hval - wval) * theta)
    cb = jnp.cos((hval + wval) * theta)
    cc = 0.5 * (ca + cb)
    ss = 0.5 * (ca - cb)
    x1 = f[:, :HALF]
    x2 = f[:, HALF:]
    ne_raw = jnp.concatenate(
        [x1 + x1 * cc - x2 * ss, x2 + x1 * ss + x2 * cc], axis=1)

    def ln(x, g, b):
        mu = jnp.mean(x, axis=1, keepdims=True)
        var = jnp.mean((x - mu) ** 2, axis=1, keepdims=True)
        return (x - mu) / jnp.sqrt(var + 1e-5) * g + b

    vecs = vecs_ref[...]
    ne = ln(ne_raw, vecs[0:1, :], vecs[1:2, :])

    def dn(a, b):
        return lax.dot_general(a, b, (((1,), (1,)), ((), ())),
                               preferred_element_type=jnp.float32)

    q = dn(ne, wq_ref[...]) + vecs[2:3, :]
    k = dn(ne, wk_ref[...]) + vecs[3:4, :]
    v = dn(ne, wv_ref[...]) + vecs[4:5, :]
    s = dn(q, k)
    s = s - jnp.max(s, axis=1, keepdims=True)
    e = jnp.exp(s)
    attw = e / jnp.sum(e, axis=1, keepdims=True)
    att = lax.dot_general(attw, v, (((1,), (0,)), ((), ())),
                          preferred_element_type=jnp.float32) + ne_raw
    h2 = ln(att, vecs[5:6, :], vecs[6:7, :])
    g1 = dn(h2, w1_ref[...]) + vecs[7:8, :]
    g1 = 0.5 * g1 * (1.0 + lax.erf(g1 * (2.0 ** -0.5)))
    upd = dn(g1, w2_ref[...]) + vecs[8:9, :] + att
    upd_ref[...] = upd.T


def _tc_compute(feats3, cpack, vecs, Wq, Wk, Wv, W1, W2):
    wspec = pl.BlockSpec((C, C), lambda b: (0, 0))
    return pl.pallas_call(
        _tc_body,
        grid=(B,),
        in_specs=[
            pl.BlockSpec((1, P, C), lambda b: (b, 0, 0)),
            pl.BlockSpec((1, P, 8), lambda b: (b, 0, 0)),
            pl.BlockSpec((16, C), lambda b: (0, 0)),
            wspec, wspec, wspec, wspec, wspec,
        ],
        out_specs=[
            pl.BlockSpec((C, P), lambda b: (b, 0)),
            pl.BlockSpec((P, 8), lambda b: (b, 0)),
        ],
        out_shape=[
            jax.ShapeDtypeStruct((B * C, P), jnp.float32),
            jax.ShapeDtypeStruct((B * P, 8), jnp.int32),
        ],
        compiler_params=pltpu.CompilerParams(
            dimension_semantics=("arbitrary",)),
    )(feats3, cpack, vecs, Wq, Wk, Wv, W1, W2)


SX = 64
SLABS = [(0, 64), (64, 64), (128, 64), (192, 64),
         (256, 64), (320, 64), (384, 48)]


def _sc_scatter(upd, ys, xs, bias16):
    mesh = plsc.VectorSubcoreMesh(core_axis_name="c", subcore_axis_name="s")

    @functools.partial(
        pl.kernel,
        mesh=mesh,
        out_type=jax.ShapeDtypeStruct((NROWS, NX, NY), jnp.float32),
        scratch_types=[
            pltpu.VMEM((8, P), jnp.float32),
            pltpu.VMEM((P,), jnp.int32),
            pltpu.VMEM((P,), jnp.int32),
            pltpu.VMEM((16,), jnp.float32),
            pltpu.VMEM((SX, NY), jnp.float32),
            pltpu.VMEM((SX, NY), jnp.float32),
            pltpu.SemaphoreType.DMA,
            pltpu.SemaphoreType.DMA,
        ],
        compiler_params=pltpu.CompilerParams(needs_layout_passes=False),
    )
    def body(upd_hbm, ys_hbm, xs_hbm, bias_hbm, out_hbm, vals_v, ys_v, xs_v,
             bias_v, buf0, buf1, sem0, sem1):
        wid = lax.axis_index("s") * 2 + lax.axis_index("c")
        row0 = wid * ROWS_PER_TILE
        grp = pl.multiple_of((wid // 2) * 8, 8)
        vo = row0 - grp
        bsel = lax.shift_right_logical(row0, 6)
        pltpu.sync_copy(ys_hbm.at[pl.ds(pl.multiple_of(bsel * P, 8), P)],
                        ys_v)
        pltpu.sync_copy(xs_hbm.at[pl.ds(pl.multiple_of(bsel * P, 8), P)],
                        xs_v)
        pltpu.sync_copy(upd_hbm.at[pl.ds(grp, 8), :], vals_v)
        pltpu.sync_copy(bias_hbm, bias_v)

        z16 = bias_v[...]

        def zinit(buf):
            def zb(x, c):
                for u in range(NY // 16):
                    buf[x, pl.ds(u * 16, 16)] = z16
                return c
            lax.fori_loop(0, SX, zb, 0)

        zinit(buf0)
        zinit(buf1)

        bufs = (buf0, buf1)
        sems = (sem0, sem1)

        def scan(buf, r_local, x0, sx, write_vals):
            vrow = vo + r_local

            def one(i):
                yv = ys_v[pl.ds(i * 16, 16)]
                xv = xs_v[pl.ds(i * 16, 16)]
                rx = xv - x0
                msk = (rx >= 0) & (rx < sx)
                rx = jnp.where(msk, rx, 0)
                if write_vals:
                    x = vals_v[vrow, pl.ds(i * 16, 16)] + z16
                else:
                    x = z16
                plsc.store_scatter(buf, [rx, yv], x, mask=msk)

            def sb(i, c):
                one(i * 2)
                one(i * 2 + 1)
                return c
            lax.fori_loop(0, P // 32, sb, 0)

        pending = [None, None]
        hist = [None, None]
        step = 0
        for r_local in range(ROWS_PER_TILE):
            row = row0 + r_local
            for (x0, sx) in SLABS:
                d = step % 2
                if pending[d] is not None:
                    pending[d].wait()
                    pr, px0, psx = hist[d]
                    scan(bufs[d], pr, px0, psx, False)
                scan(bufs[d], r_local, x0, sx, True)
                src = bufs[d] if sx == SX else bufs[d].at[pl.ds(0, sx), :]
                pending[d] = pltpu.async_copy(
                    src, out_hbm.at[row, pl.ds(x0, sx), :], sems[d])
                hist[d] = (r_local, x0, sx)
                step += 1
        pending[0].wait()
        pending[1].wait()

    return body(upd, ys, xs, bias16)


def kernel(pillar_features, voxel_coords, batch_size, ln1_g, ln1_b, Wq, bq,
           Wk, bk, Wv, bv, ln2_g, ln2_b, W1, b1, W2, b2):
    nb = pillar_features.shape[0] // P
    feats3 = pillar_features.reshape(B, P, C)
    vc = voxel_coords.astype(jnp.int32).reshape(B, P, 4)
    cpack = jnp.pad(vc, ((0, 0), (0, 0), (0, 4)))
    zeros_row = jnp.zeros_like(b2)
    vecs = jnp.stack([ln1_g, ln1_b, bq, bk, bv, ln2_g, ln2_b, b1, b2,
                      zeros_row, zeros_row, zeros_row, zeros_row,
                      zeros_row, zeros_row, zeros_row])
    upd_t, idx_o = _tc_compute(feats3, cpack, vecs, Wq, Wk, Wv, W1, W2)
    ys = idx_o[:, 0]
    xs = idx_o[:, 1]
    bias16 = jnp.full((16,), jnp.asarray(batch_size - nb, jnp.float32))
    out3 = _sc_scatter(upd_t, ys, xs, bias16)
    res = jnp.swapaxes(out3.reshape(B, C, NX, NY), 2, 3)
    return res


# speedup vs baseline: 1.1290x; 1.1290x over previous
import functools

import jax
import jax.numpy as jnp
from jax import lax
from jax.experimental import pallas as pl
from jax.experimental.pallas import tpu as pltpu
from jax.experimental.pallas import tpu_sc as plsc

NX = 432
NY = 496
C = 64
HALF = C // 2
P = 2048
B = 2
GRID_N = NY * NX
NROWS = B * C
N_TILES = 32
ROWS_PER_TILE = NROWS // N_TILES
CHUNKS_PER_ROW = 4
CHUNK = GRID_N // CHUNKS_PER_ROW
STEPS = ROWS_PER_TILE * CHUNKS_PER_ROW


def _tc_body(feats_ref, cpack_ref, vecs_ref, wq_ref, wk_ref, wv_ref,
             w1_ref, w2_ref, upd_ref, idx_ref):
    f = feats_ref[0]
    cp = cpack_ref[0]
    iz = cp[:, 1:2]
    iy = cp[:, 2:3]
    ix = cp[:, 3:4]
    ix = ix + iz
    idx_ref[...] = jnp.concatenate([iy, ix, iy, ix, iy, ix, iy, ix], axis=1)

    ar = lax.broadcasted_iota(jnp.int32, (1, HALF), 1).astype(jnp.float32)
    theta = jnp.exp(ar * (-jnp.log(10000.0) / HALF))
    hval = iy.astype(jnp.float32) * (2.0 / (NY - 1)) - 1.0
    wval = ix.astype(jnp.float32) * (2.0 / (NX - 1)) - 1.0
    arg = jnp.concatenate([(hval - wval) * theta,
                           (hval + wval) * theta], axis=1)
    cosall = jnp.cos(arg)
    ca = cosall[:, :HALF]
    cb = cosall[:, HALF:]
    cc = 0.5 * (ca + cb)
    ss = 0.5 * (ca - cb)
    x1 = f[:, :HALF]
    x2 = f[:, HALF:]
    ne_raw = jnp.concatenate(
        [x1 + x1 * cc - x2 * ss, x2 + x1 * ss + x2 * cc], axis=1)

    def ln(x, g, b):
        mu = jnp.mean(x, axis=1, keepdims=True)
        var = jnp.mean((x - mu) ** 2, axis=1, keepdims=True)
        return (x - mu) / jnp.sqrt(var + 1e-5) * g + b

    vecs = vecs_ref[...]
    ne = ln(ne_raw, vecs[0:1, :], vecs[1:2, :])

    def dn(a, b):
        return lax.dot_general(a, b, (((1,), (1,)), ((), ())),
                               preferred_element_type=jnp.float32)

    q = dn(ne, wq_ref[...]) + vecs[2:3, :]
    k = dn(ne, wk_ref[...]) + vecs[3:4, :]
    v = dn(ne, wv_ref[...]) + vecs[4:5, :]
    s = dn(q, k)
    s = s - jnp.max(s, axis=1, keepdims=True)
    e = jnp.exp(s)
    attw = e / jnp.sum(e, axis=1, keepdims=True)
    att = lax.dot_general(attw, v, (((1,), (0,)), ((), ())),
                          preferred_element_type=jnp.float32) + ne_raw
    h2 = ln(att, vecs[5:6, :], vecs[6:7, :])
    g1 = dn(h2, w1_ref[...]) + vecs[7:8, :]
    g1 = 0.5 * g1 * (1.0 + lax.erf(g1 * (2.0 ** -0.5)))
    upd = dn(g1, w2_ref[...]) + vecs[8:9, :] + att
    upd_ref[...] = upd.T


def _tc_compute(feats3, cpack, vecs, Wq, Wk, Wv, W1, W2):
    wspec = pl.BlockSpec((C, C), lambda b: (0, 0))
    return pl.pallas_call(
        _tc_body,
        grid=(B,),
        in_specs=[
            pl.BlockSpec((1, P, C), lambda b: (b, 0, 0)),
            pl.BlockSpec((1, P, 8), lambda b: (b, 0, 0)),
            pl.BlockSpec((16, C), lambda b: (0, 0)),
            wspec, wspec, wspec, wspec, wspec,
        ],
        out_specs=[
            pl.BlockSpec((C, P), lambda b: (b, 0)),
            pl.BlockSpec((P, 8), lambda b: (b, 0)),
        ],
        out_shape=[
            jax.ShapeDtypeStruct((B * C, P), jnp.float32),
            jax.ShapeDtypeStruct((B * P, 8), jnp.int32),
        ],
        compiler_params=pltpu.CompilerParams(
            dimension_semantics=("arbitrary",)),
    )(feats3, cpack, vecs, Wq, Wk, Wv, W1, W2)


SX = 64
SLABS = [(0, 64), (64, 64), (128, 64), (192, 64),
         (256, 64), (320, 64), (384, 48)]


def _sc_scatter(upd, ys, xs, bias16):
    mesh = plsc.VectorSubcoreMesh(core_axis_name="c", subcore_axis_name="s")

    @functools.partial(
        pl.kernel,
        mesh=mesh,
        out_type=jax.ShapeDtypeStruct((NROWS, NX, NY), jnp.float32),
        scratch_types=[
            pltpu.VMEM((8, P), jnp.float32),
            pltpu.VMEM((P,), jnp.int32),
            pltpu.VMEM((P,), jnp.int32),
            pltpu.VMEM((16,), jnp.float32),
            pltpu.VMEM((SX, NY), jnp.float32),
            pltpu.VMEM((SX, NY), jnp.float32),
            pltpu.SemaphoreType.DMA,
            pltpu.SemaphoreType.DMA,
        ],
        compiler_params=pltpu.CompilerParams(needs_layout_passes=False),
    )
    def body(upd_hbm, ys_hbm, xs_hbm, bias_hbm, out_hbm, vals_v, ys_v, xs_v,
             bias_v, buf0, buf1, sem0, sem1):
        wid = lax.axis_index("s") * 2 + lax.axis_index("c")
        row0 = wid * ROWS_PER_TILE
        grp = pl.multiple_of((wid // 2) * 8, 8)
        vo = row0 - grp
        bsel = lax.shift_right_logical(row0, 6)
        pltpu.sync_copy(ys_hbm.at[pl.ds(pl.multiple_of(bsel * P, 8), P)],
                        ys_v)
        pltpu.sync_copy(xs_hbm.at[pl.ds(pl.multiple_of(bsel * P, 8), P)],
                        xs_v)
        pltpu.sync_copy(upd_hbm.at[pl.ds(grp, 8), :], vals_v)
        pltpu.sync_copy(bias_hbm, bias_v)

        z16 = bias_v[...]

        def zinit(buf):
            def zb(x, c):
                for u in range(NY // 16):
                    buf[x, pl.ds(u * 16, 16)] = z16
                return c
            lax.fori_loop(0, SX, zb, 0)

        zinit(buf0)
        zinit(buf1)

        bufs = (buf0, buf1)
        sems = (sem0, sem1)

        def scan(buf, r_local, x0, sx, write_vals):
            vrow = vo + r_local

            def one(i):
                yv = ys_v[pl.ds(i * 16, 16)]
                xv = xs_v[pl.ds(i * 16, 16)]
                rx = xv - x0
                msk = (rx >= 0) & (rx < sx)
                rx = jnp.where(msk, rx, 0)
                if write_vals:
                    x = vals_v[vrow, pl.ds(i * 16, 16)] + z16
                else:
                    x = z16
                plsc.store_scatter(buf, [rx, yv], x, mask=msk)

            def sb(i, c):
                one(i * 2)
                one(i * 2 + 1)
                return c
            lax.fori_loop(0, P // 32, sb, 0)

        pending = [None, None]
        hist = [None, None]
        step = 0
        for r_local in range(ROWS_PER_TILE):
            row = row0 + r_local
            for (x0, sx) in SLABS:
                d = step % 2
                if pending[d] is not None:
                    pending[d].wait()
                    pr, px0, psx = hist[d]
                    scan(bufs[d], pr, px0, psx, False)
                scan(bufs[d], r_local, x0, sx, True)
                src = bufs[d] if sx == SX else bufs[d].at[pl.ds(0, sx), :]
                pending[d] = pltpu.async_copy(
                    src, out_hbm.at[row, pl.ds(x0, sx), :], sems[d])
                hist[d] = (r_local, x0, sx)
                step += 1
        pending[0].wait()
        pending[1].wait()

    return body(upd, ys, xs, bias16)


def kernel(pillar_features, voxel_coords, batch_size, ln1_g, ln1_b, Wq, bq,
           Wk, bk, Wv, bv, ln2_g, ln2_b, W1, b1, W2, b2):
    nb = pillar_features.shape[0] // P
    feats3 = pillar_features.reshape(B, P, C)
    vc = voxel_coords.astype(jnp.int32).reshape(B, P, 4)
    cpack = jnp.pad(vc, ((0, 0), (0, 0), (0, 4)))
    zeros_row = jnp.zeros_like(b2)
    vecs = jnp.stack([ln1_g, ln1_b, bq, bk, bv, ln2_g, ln2_b, b1, b2,
                      zeros_row, zeros_row, zeros_row, zeros_row,
                      zeros_row, zeros_row, zeros_row])
    upd_t, idx_o = _tc_compute(feats3, cpack, vecs, Wq, Wk, Wv, W1, W2)
    ys = idx_o[:, 0]
    xs = idx_o[:, 1]
    bias16 = jnp.full((16,), jnp.asarray(batch_size - nb, jnp.float32))
    out3 = _sc_scatter(upd_t, ys, xs, bias16)
    res = jnp.swapaxes(out3.reshape(B, C, NX, NY), 2, 3)
    return res


# speedup vs baseline: 1.1299x; 1.0008x over previous
import functools

import jax
import jax.numpy as jnp
from jax import lax
from jax.experimental import pallas as pl
from jax.experimental.pallas import tpu as pltpu
from jax.experimental.pallas import tpu_sc as plsc

NX = 432
NY = 496
C = 64
HALF = C // 2
P = 2048
B = 2
GRID_N = NY * NX
NROWS = B * C
N_TILES = 32
ROWS_PER_TILE = NROWS // N_TILES
CHUNKS_PER_ROW = 4
CHUNK = GRID_N // CHUNKS_PER_ROW
STEPS = ROWS_PER_TILE * CHUNKS_PER_ROW


def _tc_body(feats_ref, cpack_ref, vecs_ref, wq_ref, wk_ref, wv_ref,
             w1_ref, w2_ref, upd_ref, idx_ref):
    f = feats_ref[0]
    cp = cpack_ref[0]
    iz = cp[:, 1:2]
    iy = cp[:, 2:3]
    ix = cp[:, 3:4]
    ix = ix + iz
    idx_ref[...] = jnp.concatenate([iy, ix, iy, ix, iy, ix, iy, ix], axis=1)

    ar = lax.broadcasted_iota(jnp.int32, (1, HALF), 1).astype(jnp.float32)
    theta = jnp.exp(ar * (-jnp.log(10000.0) / HALF))
    hval = iy.astype(jnp.float32) * (2.0 / (NY - 1)) - 1.0
    wval = ix.astype(jnp.float32) * (2.0 / (NX - 1)) - 1.0
    arg = jnp.concatenate([(hval - wval) * theta,
                           (hval + wval) * theta], axis=1)
    cosall = jnp.cos(arg)
    ca = cosall[:, :HALF]
    cb = cosall[:, HALF:]
    cc = 0.5 * (ca + cb)
    ss = 0.5 * (ca - cb)
    x1 = f[:, :HALF]
    x2 = f[:, HALF:]
    ne_raw = jnp.concatenate(
        [x1 + x1 * cc - x2 * ss, x2 + x1 * ss + x2 * cc], axis=1)

    def ln(x, g, b):
        mu = jnp.mean(x, axis=1, keepdims=True)
        var = jnp.mean((x - mu) ** 2, axis=1, keepdims=True)
        return (x - mu) / jnp.sqrt(var + 1e-5) * g + b

    vecs = vecs_ref[...]
    ne = ln(ne_raw, vecs[0:1, :], vecs[1:2, :])

    def dn(a, b):
        return lax.dot_general(a, b, (((1,), (1,)), ((), ())),
                               preferred_element_type=jnp.float32)

    q = dn(ne, wq_ref[...]) + vecs[2:3, :]
    k = dn(ne, wk_ref[...]) + vecs[3:4, :]
    v = dn(ne, wv_ref[...]) + vecs[4:5, :]
    s = dn(q, k)
    s = s - jnp.max(s, axis=1, keepdims=True)
    e = jnp.exp(s)
    attw = e / jnp.sum(e, axis=1, keepdims=True)
    att = lax.dot_general(attw, v, (((1,), (0,)), ((), ())),
                          preferred_element_type=jnp.float32) + ne_raw
    h2 = ln(att, vecs[5:6, :], vecs[6:7, :])
    g1 = dn(h2, w1_ref[...]) + vecs[7:8, :]
    g1 = 0.5 * g1 * (1.0 + lax.erf(g1 * (2.0 ** -0.5)))
    upd = dn(g1, w2_ref[...]) + vecs[8:9, :] + att
    upd_ref[...] = upd.T


def _tc_compute(feats3, cpack, vecs, Wq, Wk, Wv, W1, W2):
    wspec = pl.BlockSpec((C, C), lambda b: (0, 0))
    return pl.pallas_call(
        _tc_body,
        grid=(B,),
        in_specs=[
            pl.BlockSpec((1, P, C), lambda b: (b, 0, 0)),
            pl.BlockSpec((1, P, 8), lambda b: (b, 0, 0)),
            pl.BlockSpec((16, C), lambda b: (0, 0)),
            wspec, wspec, wspec, wspec, wspec,
        ],
        out_specs=[
            pl.BlockSpec((C, P), lambda b: (b, 0)),
            pl.BlockSpec((P, 8), lambda b: (b, 0)),
        ],
        out_shape=[
            jax.ShapeDtypeStruct((B * C, P), jnp.float32),
            jax.ShapeDtypeStruct((B * P, 8), jnp.int32),
        ],
        compiler_params=pltpu.CompilerParams(
            dimension_semantics=("arbitrary",)),
    )(feats3, cpack, vecs, Wq, Wk, Wv, W1, W2)


SX = 64
SLABS = [(0, 64), (64, 64), (128, 64), (192, 64),
         (256, 64), (320, 64), (384, 48)]


def _sc_scatter(upd, ys, xs, bias16):
    mesh = plsc.VectorSubcoreMesh(core_axis_name="c", subcore_axis_name="s")

    @functools.partial(
        pl.kernel,
        mesh=mesh,
        out_type=jax.ShapeDtypeStruct((NROWS, NX, NY), jnp.float32),
        scratch_types=[
            pltpu.VMEM((8, P), jnp.float32),
            pltpu.VMEM((P,), jnp.int32),
            pltpu.VMEM((P,), jnp.int32),
            pltpu.VMEM((16,), jnp.float32),
            pltpu.VMEM((SX, NY), jnp.float32),
            pltpu.VMEM((SX, NY), jnp.float32),
            pltpu.VMEM((SX, NY), jnp.float32),
            pltpu.SemaphoreType.DMA,
            pltpu.SemaphoreType.DMA,
            pltpu.SemaphoreType.DMA,
        ],
        compiler_params=pltpu.CompilerParams(needs_layout_passes=False),
    )
    def body(upd_hbm, ys_hbm, xs_hbm, bias_hbm, out_hbm, vals_v, ys_v, xs_v,
             bias_v, buf0, buf1, buf2, sem0, sem1, sem2):
        wid = lax.axis_index("s") * 2 + lax.axis_index("c")
        row0 = wid * ROWS_PER_TILE
        grp = pl.multiple_of((wid // 2) * 8, 8)
        vo = row0 - grp
        bsel = lax.shift_right_logical(row0, 6)
        pltpu.sync_copy(ys_hbm.at[pl.ds(pl.multiple_of(bsel * P, 8), P)],
                        ys_v)
        pltpu.sync_copy(xs_hbm.at[pl.ds(pl.multiple_of(bsel * P, 8), P)],
                        xs_v)
        pltpu.sync_copy(upd_hbm.at[pl.ds(grp, 8), :], vals_v)
        pltpu.sync_copy(bias_hbm, bias_v)

        z16 = bias_v[...]

        def zinit(buf):
            def zb(x, c):
                for u in range(NY // 16):
                    buf[x, pl.ds(u * 16, 16)] = z16
                return c
            lax.fori_loop(0, SX, zb, 0)

        zinit(buf0)
        zinit(buf1)
        zinit(buf2)

        bufs = (buf0, buf1, buf2)
        sems = (sem0, sem1, sem2)

        def scan(buf, r_local, x0, sx, write_vals):
            vrow = vo + r_local

            def one(i):
                yv = ys_v[pl.ds(i * 16, 16)]
                xv = xs_v[pl.ds(i * 16, 16)]
                rx = xv - x0
                msk = (rx >= 0) & (rx < sx)
                rx = jnp.where(msk, rx, 0)
                if write_vals:
                    x = vals_v[vrow, pl.ds(i * 16, 16)] + z16
                else:
                    x = z16
                plsc.store_scatter(buf, [rx, yv], x, mask=msk)

            def sb(i, c):
                one(i * 2)
                one(i * 2 + 1)
                return c
            lax.fori_loop(0, P // 32, sb, 0)

        pending = [None, None, None]
        hist = [None, None, None]
        step = 0
        for r_local in range(ROWS_PER_TILE):
            row = row0 + r_local
            for (x0, sx) in SLABS:
                d = step % 3
                if pending[d] is not None:
                    pending[d].wait()
                    pr, px0, psx = hist[d]
                    scan(bufs[d], pr, px0, psx, False)
                scan(bufs[d], r_local, x0, sx, True)
                src = bufs[d] if sx == SX else bufs[d].at[pl.ds(0, sx), :]
                pending[d] = pltpu.async_copy(
                    src, out_hbm.at[row, pl.ds(x0, sx), :], sems[d])
                hist[d] = (r_local, x0, sx)
                step += 1
        pending[0].wait()
        pending[1].wait()
        pending[2].wait()

    return body(upd, ys, xs, bias16)


def kernel(pillar_features, voxel_coords, batch_size, ln1_g, ln1_b, Wq, bq,
           Wk, bk, Wv, bv, ln2_g, ln2_b, W1, b1, W2, b2):
    nb = pillar_features.shape[0] // P
    feats3 = pillar_features.reshape(B, P, C)
    vc = voxel_coords.astype(jnp.int32).reshape(B, P, 4)
    cpack = jnp.pad(vc, ((0, 0), (0, 0), (0, 4)))
    zeros_row = jnp.zeros_like(b2)
    vecs = jnp.stack([ln1_g, ln1_b, bq, bk, bv, ln2_g, ln2_b, b1, b2,
                      zeros_row, zeros_row, zeros_row, zeros_row,
                      zeros_row, zeros_row, zeros_row])
    upd_t, idx_o = _tc_compute(feats3, cpack, vecs, Wq, Wk, Wv, W1, W2)
    ys = idx_o[:, 0]
    xs = idx_o[:, 1]
    bias16 = jnp.full((16,), jnp.asarray(batch_size - nb, jnp.float32))
    out3 = _sc_scatter(upd_t, ys, xs, bias16)
    res = jnp.swapaxes(out3.reshape(B, C, NX, NY), 2, 3)
    return res
